# baseline (speedup 1.0000x reference)
import functools

import jax
import jax.numpy as jnp
from jax import lax
from jax.experimental import pallas as pl
from jax.experimental.pallas import tpu as pltpu
from jax.experimental.pallas import tpu_sc as plsc

_NC = 2
_NS = 16
_NW = _NC * _NS
_LANES = 16
_NCHUNK = 1


def _gather_norm_fn(b_per_w, table_hbm, idx_hbm, mean_hbm, norm_hbm, out_hbm,
                    idx_v, vals_v, mean_v, norm_v, *sems):
    sem_i = sems[:_NCHUNK]
    sem_g = sems[_NCHUNK:2 * _NCHUNK]
    sem_o = sems[2 * _NCHUNK:3 * _NCHUNK]
    sem_m, sem_n = sems[3 * _NCHUNK:]
    wid = lax.axis_index("s") * _NC + lax.axis_index("c")
    base = wid * b_per_w
    ck = b_per_w // _NCHUNK

    i_cps = [pltpu.async_copy(idx_hbm.at[pl.ds(base + c * ck, ck)],
                              idx_v.at[pl.ds(c * ck, ck)], sem_i[c])
             for c in range(_NCHUNK)]
    zeros = jnp.zeros((_LANES,), jnp.int32)
    m_cp = pltpu.async_copy(mean_hbm.at[zeros], mean_v, sem_m)
    n_cp = pltpu.async_copy(norm_hbm.at[zeros], norm_v, sem_n)

    g_cps = []
    for c in range(_NCHUNK):
        i_cps[c].wait()
        g_cps.append(
            pltpu.async_copy(table_hbm.at[idx_v.at[pl.ds(c * ck, ck)]],
                             vals_v.at[pl.ds(c * ck, ck)], sem_g[c]))
    m_cp.wait()
    n_cp.wait()
    scale = 1.0 / norm_v[...]
    shift = mean_v[...] * scale

    o_cps = []
    for c in range(_NCHUNK):
        g_cps[c].wait()

        @pl.loop(c * ck, (c + 1) * ck, step=_LANES)
        def _(i):
            vals_v[pl.ds(i, _LANES)] = vals_v[pl.ds(i, _LANES)] * scale - shift

        o_cps.append(
            pltpu.async_copy(vals_v.at[pl.ds(c * ck, ck)],
                             out_hbm.at[pl.ds(base + c * ck, ck)], sem_o[c]))
    for o_cp in o_cps:
        o_cp.wait()


def kernel(inputs, categ_bias, moving_mean, moving_norm):
    batch = inputs.shape[0]
    idx = inputs.reshape(batch).astype(jnp.int32)
    table = categ_bias.reshape(-1)
    b_per_w = batch // _NW

    mesh = plsc.VectorSubcoreMesh(core_axis_name="c", subcore_axis_name="s")
    run = pl.kernel(
        functools.partial(_gather_norm_fn, b_per_w),
        out_type=jax.ShapeDtypeStruct((batch,), jnp.float32),
        mesh=mesh,
        scratch_types=[
            pltpu.VMEM((b_per_w,), jnp.int32),
            pltpu.VMEM((b_per_w,), jnp.float32),
            pltpu.VMEM((_LANES,), jnp.float32),
            pltpu.VMEM((_LANES,), jnp.float32),
        ] + [pltpu.SemaphoreType.DMA] * (3 * _NCHUNK + 2),
    )
    out = run(table, idx, moving_mean.astype(jnp.float32),
              moving_norm.astype(jnp.float32))
    return out.reshape(batch, 1)


# speedup vs baseline: 1.0116x; 1.0116x over previous
import functools

import jax
import jax.numpy as jnp
from jax import lax
from jax.experimental import pallas as pl
from jax.experimental.pallas import tpu as pltpu
from jax.experimental.pallas import tpu_sc as plsc

_NC = 2
_NS = 16
_NW = _NC * _NS
_LANES = 16
_NCHUNK = 2
_NSUB = 2


def _gather_norm_fn(b_per_w, table_hbm, idx_hbm, mean_hbm, norm_hbm, out_hbm,
                    idx_v, vals_v, mean_v, norm_v, *sems):
    sem_i = sems[:_NCHUNK]
    sem_g = sems[_NCHUNK:2 * _NCHUNK]
    sem_o = sems[2 * _NCHUNK:2 * _NCHUNK + _NCHUNK * _NSUB]
    sem_m, sem_n = sems[2 * _NCHUNK + _NCHUNK * _NSUB:]
    wid = lax.axis_index("s") * _NC + lax.axis_index("c")
    base = wid * b_per_w
    ck = b_per_w // _NCHUNK

    i_cps = [pltpu.async_copy(idx_hbm.at[pl.ds(base + c * ck, ck)],
                              idx_v.at[pl.ds(c * ck, ck)], sem_i[c])
             for c in range(_NCHUNK)]
    zeros = jnp.zeros((_LANES,), jnp.int32)
    m_cp = pltpu.async_copy(mean_hbm.at[zeros], mean_v, sem_m)
    n_cp = pltpu.async_copy(norm_hbm.at[zeros], norm_v, sem_n)

    g_cps = []
    for c in range(_NCHUNK):
        i_cps[c].wait()
        g_cps.append(
            pltpu.async_copy(table_hbm.at[idx_v.at[pl.ds(c * ck, ck)]],
                             vals_v.at[pl.ds(c * ck, ck)], sem_g[c]))
    m_cp.wait()
    n_cp.wait()
    scale = 1.0 / norm_v[...]
    shift = mean_v[...] * scale

    sub = ck // _NSUB
    o_cps = []
    for c in range(_NCHUNK):
        g_cps[c].wait()
        for s in range(_NSUB):
            lo = c * ck + s * sub

            @pl.loop(lo, lo + sub, step=_LANES)
            def _(i):
                vals_v[pl.ds(i, _LANES)] = (
                    vals_v[pl.ds(i, _LANES)] * scale - shift)

            o_cps.append(
                pltpu.async_copy(vals_v.at[pl.ds(lo, sub)],
                                 out_hbm.at[pl.ds(base + lo, sub)],
                                 sem_o[c * _NSUB + s]))
    for o_cp in o_cps:
        o_cp.wait()


def kernel(inputs, categ_bias, moving_mean, moving_norm):
    batch = inputs.shape[0]
    idx = inputs.reshape(batch).astype(jnp.int32)
    table = categ_bias.reshape(-1)
    b_per_w = batch // _NW

    mesh = plsc.VectorSubcoreMesh(core_axis_name="c", subcore_axis_name="s")
    run = pl.kernel(
        functools.partial(_gather_norm_fn, b_per_w),
        out_type=jax.ShapeDtypeStruct((batch,), jnp.float32),
        mesh=mesh,
        scratch_types=[
            pltpu.VMEM((b_per_w,), jnp.int32),
            pltpu.VMEM((b_per_w,), jnp.float32),
            pltpu.VMEM((_LANES,), jnp.float32),
            pltpu.VMEM((_LANES,), jnp.float32),
        ] + [pltpu.SemaphoreType.DMA] * (2 * _NCHUNK + _NCHUNK * _NSUB + 2),
    )
    out = run(table, idx, moving_mean.astype(jnp.float32),
              moving_norm.astype(jnp.float32))
    return out.reshape(batch, 1)


# speedup vs baseline: 1.0233x; 1.0116x over previous
import functools

import jax
import jax.numpy as jnp
from jax import lax
from jax.experimental import pallas as pl
from jax.experimental.pallas import tpu as pltpu
from jax.experimental.pallas import tpu_sc as plsc

_NC = 2
_NS = 16
_NW = _NC * _NS
_LANES = 16
_NCHUNK = 2
_NSUB = 1


def _gather_norm_fn(b_per_w, table_hbm, idx_hbm, mean_hbm, norm_hbm, out_hbm,
                    idx_v, vals_v, mean_v, norm_v, *sems):
    sem_i = sems[:_NCHUNK]
    sem_g = sems[_NCHUNK:2 * _NCHUNK]
    sem_o = sems[2 * _NCHUNK:2 * _NCHUNK + _NCHUNK * _NSUB]
    sem_m, sem_n = sems[2 * _NCHUNK + _NCHUNK * _NSUB:]
    wid = lax.axis_index("s") * _NC + lax.axis_index("c")
    base = wid * b_per_w
    ck = b_per_w // _NCHUNK

    i_cps = [pltpu.async_copy(idx_hbm.at[pl.ds(base + c * ck, ck)],
                              idx_v.at[pl.ds(c * ck, ck)], sem_i[c])
             for c in range(_NCHUNK)]
    zeros = jnp.zeros((_LANES,), jnp.int32)
    m_cp = pltpu.async_copy(mean_hbm.at[zeros], mean_v, sem_m)
    n_cp = pltpu.async_copy(norm_hbm.at[zeros], norm_v, sem_n)

    g_cps = []
    for c in range(_NCHUNK):
        i_cps[c].wait()
        g_cps.append(
            pltpu.async_copy(table_hbm.at[idx_v.at[pl.ds(c * ck, ck)]],
                             vals_v.at[pl.ds(c * ck, ck)], sem_g[c]))
    m_cp.wait()
    n_cp.wait()
    scale = 1.0 / norm_v[...]
    shift = mean_v[...] * scale

    sub = ck // _NSUB
    o_cps = []
    for c in range(_NCHUNK):
        g_cps[c].wait()
        for s in range(_NSUB):
            lo = c * ck + s * sub
            for i in range(lo, lo + sub, _LANES):
                vals_v[pl.ds(i, _LANES)] = (
                    vals_v[pl.ds(i, _LANES)] * scale - shift)

            o_cps.append(
                pltpu.async_copy(vals_v.at[pl.ds(lo, sub)],
                                 out_hbm.at[pl.ds(base + lo, sub)],
                                 sem_o[c * _NSUB + s]))
    for o_cp in o_cps:
        o_cp.wait()


def kernel(inputs, categ_bias, moving_mean, moving_norm):
    batch = inputs.shape[0]
    idx = inputs.reshape(batch).astype(jnp.int32)
    table = categ_bias.reshape(-1)
    b_per_w = batch // _NW

    mesh = plsc.VectorSubcoreMesh(core_axis_name="c", subcore_axis_name="s")
    run = pl.kernel(
        functools.partial(_gather_norm_fn, b_per_w),
        out_type=jax.ShapeDtypeStruct((batch,), jnp.float32),
        mesh=mesh,
        scratch_types=[
            pltpu.VMEM((b_per_w,), jnp.int32),
            pltpu.VMEM((b_per_w,), jnp.float32),
            pltpu.VMEM((_LANES,), jnp.float32),
            pltpu.VMEM((_LANES,), jnp.float32),
        ] + [pltpu.SemaphoreType.DMA] * (2 * _NCHUNK + _NCHUNK * _NSUB + 2),
    )
    out = run(table, idx, moving_mean.astype(jnp.float32),
              moving_norm.astype(jnp.float32))
    return out.reshape(batch, 1)


# speedup vs baseline: 1.0253x; 1.0019x over previous
import functools

import jax
import jax.numpy as jnp
from jax import lax
from jax.experimental import pallas as pl
from jax.experimental.pallas import tpu as pltpu
from jax.experimental.pallas import tpu_sc as plsc

_NC = 2
_NS = 16
_NW = _NC * _NS
_LANES = 16
_SPLITS = (192, 320)


def _gather_norm_fn(b_per_w, table_hbm, idx_hbm, mean_hbm, norm_hbm, out_hbm,
                    idx_v, vals_v, mean_v, norm_v, *sems):
    nchunk = len(_SPLITS)
    sem_i = sems[:nchunk]
    sem_g = sems[nchunk:2 * nchunk]
    sem_o = sems[2 * nchunk:3 * nchunk]
    sem_m, sem_n = sems[3 * nchunk:]
    wid = lax.axis_index("s") * _NC + lax.axis_index("c")
    base = wid * b_per_w
    offs = [sum(_SPLITS[:c]) for c in range(nchunk)]

    i_cps = [pltpu.async_copy(idx_hbm.at[pl.ds(base + offs[c], _SPLITS[c])],
                              idx_v.at[pl.ds(offs[c], _SPLITS[c])], sem_i[c])
             for c in range(nchunk)]
    zeros = jnp.zeros((_LANES,), jnp.int32)
    m_cp = pltpu.async_copy(mean_hbm.at[zeros], mean_v, sem_m)
    n_cp = pltpu.async_copy(norm_hbm.at[zeros], norm_v, sem_n)

    g_cps = []
    for c in range(nchunk):
        i_cps[c].wait()
        g_cps.append(
            pltpu.async_copy(table_hbm.at[idx_v.at[pl.ds(offs[c], _SPLITS[c])]],
                             vals_v.at[pl.ds(offs[c], _SPLITS[c])], sem_g[c]))
    m_cp.wait()
    n_cp.wait()
    scale = 1.0 / norm_v[...]
    shift = mean_v[...] * scale

    o_cps = []
    for c in range(nchunk):
        g_cps[c].wait()
        for i in range(offs[c], offs[c] + _SPLITS[c], _LANES):
            vals_v[pl.ds(i, _LANES)] = (
                vals_v[pl.ds(i, _LANES)] * scale - shift)

        o_cps.append(
            pltpu.async_copy(vals_v.at[pl.ds(offs[c], _SPLITS[c])],
                             out_hbm.at[pl.ds(base + offs[c], _SPLITS[c])],
                             sem_o[c]))
    for o_cp in o_cps:
        o_cp.wait()


def kernel(inputs, categ_bias, moving_mean, moving_norm):
    batch = inputs.shape[0]
    idx = inputs.reshape(batch).astype(jnp.int32)
    table = categ_bias.reshape(-1)
    b_per_w = batch // _NW

    mesh = plsc.VectorSubcoreMesh(core_axis_name="c", subcore_axis_name="s")
    run = pl.kernel(
        functools.partial(_gather_norm_fn, b_per_w),
        out_type=jax.ShapeDtypeStruct((batch,), jnp.float32),
        mesh=mesh,
        scratch_types=[
            pltpu.VMEM((b_per_w,), jnp.int32),
            pltpu.VMEM((b_per_w,), jnp.float32),
            pltpu.VMEM((_LANES,), jnp.float32),
            pltpu.VMEM((_LANES,), jnp.float32),
        ] + [pltpu.SemaphoreType.DMA] * (3 * len(_SPLITS) + 2),
    )
    out = run(table, idx, moving_mean.astype(jnp.float32),
              moving_norm.astype(jnp.float32))
    return out.reshape(batch, 1)
